# baseline (speedup 1.0000x reference)
import functools

import jax
import jax.numpy as jnp
from jax.experimental import pallas as pl
from jax.experimental.pallas import tpu as pltpu


def _row_tile(n: int) -> int:
    for t in (400, 200, 40, 8):
        if n % t == 0:
            return t
    return n


def _matmul_body(x_ref, w_ref, o_ref):
    o_ref[...] = jnp.dot(x_ref[...], w_ref[...], preferred_element_type=jnp.float32)


def _input_proj(x, w):
    n = x.shape[0]
    g = w.shape[1]
    return pl.pallas_call(
        _matmul_body,
        out_shape=jax.ShapeDtypeStruct((n, g), jnp.float32),
    )(x, w)


def _layer_body(adj_ref, h_ref, b_ref, *rest, relu, fused):
    if fused:
        wn_ref, o_ref = rest
    else:
        (o_ref,) = rest
    y = jnp.dot(adj_ref[...], h_ref[...], preferred_element_type=jnp.float32)
    y = y + b_ref[...]
    if relu:
        y = jnp.maximum(y, 0.0)
    if fused:
        y = jnp.dot(y, wn_ref[...], preferred_element_type=jnp.float32)
    o_ref[...] = y


def _layer(adj, h, b, w_next=None, relu=True):
    n = adj.shape[0]
    g = h.shape[1]
    gout = w_next.shape[1] if w_next is not None else g
    bm = _row_tile(n)
    fused = w_next is not None
    args = [adj, h, b.reshape(1, -1)]
    in_specs = [
        pl.BlockSpec((bm, n), lambda i: (i, 0)),
        pl.BlockSpec((n, g), lambda i: (0, 0)),
        pl.BlockSpec((1, g), lambda i: (0, 0)),
    ]
    if fused:
        args.append(w_next)
        in_specs.append(pl.BlockSpec((g, gout), lambda i: (0, 0)))
    return pl.pallas_call(
        functools.partial(_layer_body, relu=relu, fused=fused),
        grid=(n // bm,),
        in_specs=in_specs,
        out_specs=pl.BlockSpec((bm, gout), lambda i: (i, 0)),
        out_shape=jax.ShapeDtypeStruct((n, gout), jnp.float32),
        compiler_params=pltpu.CompilerParams(
            dimension_semantics=("parallel",)),
    )(*args)


def _gram_body(z_ref, zt_ref, o_ref):
    o_ref[...] = jnp.dot(z_ref[...], zt_ref[...], preferred_element_type=jnp.float32)


def _gram(z):
    n, g = z.shape
    bm = _row_tile(n)
    zt = z.T
    return pl.pallas_call(
        _gram_body,
        grid=(n // bm,),
        in_specs=[
            pl.BlockSpec((bm, g), lambda i: (i, 0)),
            pl.BlockSpec((g, n), lambda i: (0, 0)),
        ],
        out_specs=pl.BlockSpec((bm, n), lambda i: (i, 0)),
        out_shape=jax.ShapeDtypeStruct((n, n), jnp.float32),
        compiler_params=pltpu.CompilerParams(
            dimension_semantics=("parallel",)),
    )(z, zt)


def kernel(feat, adj, W1, b1, W2, b2, W3, b3):
    h1 = _input_proj(feat, W1)
    h2 = _layer(adj, h1, b1, w_next=W2, relu=True)
    h3 = _layer(adj, h2, b2, w_next=W3, relu=True)
    z = _layer(adj, h3, b3, w_next=None, relu=False)
    return _gram(z)


# speedup vs baseline: 1.0654x; 1.0654x over previous
import functools

import jax
import jax.numpy as jnp
from jax.experimental import pallas as pl
from jax.experimental.pallas import tpu as pltpu


def _row_tile(n: int, target: int) -> int:
    for t in range(target, 0, -1):
        if n % t == 0 and t % 8 == 0:
            return t
    return n


def _matmul_body(x_ref, w_ref, o_ref):
    h = jnp.dot(x_ref[...], w_ref[...], preferred_element_type=jnp.float32)
    o_ref[...] = h.astype(o_ref.dtype)


def _input_proj(x, w):
    n = x.shape[0]
    g = w.shape[1]
    return pl.pallas_call(
        _matmul_body,
        out_shape=jax.ShapeDtypeStruct((n, g), jnp.bfloat16),
    )(x, w)


def _layer1_body(adj_ref, h_ref, b_ref, wn_ref, o_ref, adj16_ref):
    a16 = adj_ref[...].astype(jnp.bfloat16)
    adj16_ref[...] = a16
    y = jnp.dot(a16, h_ref[...], preferred_element_type=jnp.float32)
    y = jnp.maximum(y + b_ref[...], 0.0)
    h2 = jnp.dot(y, wn_ref[...], preferred_element_type=jnp.float32)
    o_ref[...] = h2.astype(jnp.bfloat16)


def _layer1(adj, h, b, w_next):
    n = adj.shape[0]
    g = h.shape[1]
    gout = w_next.shape[1]
    bm = _row_tile(n, 200)
    return pl.pallas_call(
        _layer1_body,
        grid=(n // bm,),
        in_specs=[
            pl.BlockSpec((bm, n), lambda i: (i, 0)),
            pl.BlockSpec((n, g), lambda i: (0, 0)),
            pl.BlockSpec((1, g), lambda i: (0, 0)),
            pl.BlockSpec((g, gout), lambda i: (0, 0)),
        ],
        out_specs=[
            pl.BlockSpec((bm, gout), lambda i: (i, 0)),
            pl.BlockSpec((bm, n), lambda i: (i, 0)),
        ],
        out_shape=[
            jax.ShapeDtypeStruct((n, gout), jnp.bfloat16),
            jax.ShapeDtypeStruct((n, n), jnp.bfloat16),
        ],
        compiler_params=pltpu.CompilerParams(
            dimension_semantics=("parallel",)),
    )(adj, h, b.reshape(1, -1), w_next)


def _layer_body(adj_ref, h_ref, b_ref, *rest, relu, fused):
    if fused:
        wn_ref, o_ref = rest
    else:
        (o_ref,) = rest
    y = jnp.dot(adj_ref[...], h_ref[...], preferred_element_type=jnp.float32)
    y = y + b_ref[...]
    if relu:
        y = jnp.maximum(y, 0.0)
    if fused:
        y = jnp.dot(y, wn_ref[...], preferred_element_type=jnp.float32)
    o_ref[...] = y.astype(o_ref.dtype)


def _layer(adj16, h, b, w_next=None, relu=True, out_dtype=jnp.float32):
    n = adj16.shape[0]
    g = h.shape[1]
    gout = w_next.shape[1] if w_next is not None else g
    bm = _row_tile(n, 400)
    fused = w_next is not None
    args = [adj16, h, b.reshape(1, -1)]
    in_specs = [
        pl.BlockSpec((bm, n), lambda i: (i, 0)),
        pl.BlockSpec((n, g), lambda i: (0, 0)),
        pl.BlockSpec((1, g), lambda i: (0, 0)),
    ]
    if fused:
        args.append(w_next)
        in_specs.append(pl.BlockSpec((g, gout), lambda i: (0, 0)))
    return pl.pallas_call(
        functools.partial(_layer_body, relu=relu, fused=fused),
        grid=(n // bm,),
        in_specs=in_specs,
        out_specs=pl.BlockSpec((bm, gout), lambda i: (i, 0)),
        out_shape=jax.ShapeDtypeStruct((n, gout), out_dtype),
        compiler_params=pltpu.CompilerParams(
            dimension_semantics=("parallel",)),
    )(*args)


def _gram_body(z_ref, zt_ref, o_ref):
    o_ref[...] = jnp.dot(z_ref[...], zt_ref[...], preferred_element_type=jnp.float32)


def _gram(z):
    n, g = z.shape
    bm = _row_tile(n, 400)
    zt = z.T
    return pl.pallas_call(
        _gram_body,
        grid=(n // bm,),
        in_specs=[
            pl.BlockSpec((bm, g), lambda i: (i, 0)),
            pl.BlockSpec((g, n), lambda i: (0, 0)),
        ],
        out_specs=pl.BlockSpec((bm, n), lambda i: (i, 0)),
        out_shape=jax.ShapeDtypeStruct((n, n), jnp.float32),
        compiler_params=pltpu.CompilerParams(
            dimension_semantics=("parallel",)),
    )(z, zt)


def kernel(feat, adj, W1, b1, W2, b2, W3, b3):
    h1 = _input_proj(feat, W1)
    h2, adj16 = _layer1(adj, h1, b1, W2)
    h3 = _layer(adj16, h2, b2, w_next=W3, relu=True, out_dtype=jnp.bfloat16)
    z = _layer(adj16, h3, b3, w_next=None, relu=False, out_dtype=jnp.float32)
    return _gram(z)


# speedup vs baseline: 1.2356x; 1.1597x over previous
import functools

import jax
import jax.numpy as jnp
from jax.experimental import pallas as pl
from jax.experimental.pallas import tpu as pltpu


def _row_tile(n: int, target: int) -> int:
    for t in range(target, 0, -1):
        if n % t == 0 and t % 8 == 0:
            return t
    return n


def _matmul_body(x_ref, w_ref, o_ref):
    h = jnp.dot(x_ref[...], w_ref[...], preferred_element_type=jnp.float32)
    o_ref[...] = h.astype(o_ref.dtype)


def _input_proj(x, w):
    n = x.shape[0]
    g = w.shape[1]
    return pl.pallas_call(
        _matmul_body,
        out_shape=jax.ShapeDtypeStruct((n, g), jnp.bfloat16),
    )(x, w)


def _layer1_body(adj_ref, h_ref, b_ref, wn_ref, o_ref, adj8_ref):
    a16 = adj_ref[...].astype(jnp.bfloat16)
    adj8_ref[...] = adj_ref[...].astype(jnp.float8_e4m3fn)
    y = jnp.dot(a16, h_ref[...], preferred_element_type=jnp.float32)
    y = jnp.maximum(y + b_ref[...], 0.0)
    h2 = jnp.dot(y, wn_ref[...], preferred_element_type=jnp.float32)
    o_ref[...] = h2.astype(jnp.bfloat16)


def _layer1(adj, h, b, w_next):
    n = adj.shape[0]
    g = h.shape[1]
    gout = w_next.shape[1]
    bm = _row_tile(n, 200)
    return pl.pallas_call(
        _layer1_body,
        grid=(n // bm,),
        in_specs=[
            pl.BlockSpec((bm, n), lambda i: (i, 0)),
            pl.BlockSpec((n, g), lambda i: (0, 0)),
            pl.BlockSpec((1, g), lambda i: (0, 0)),
            pl.BlockSpec((g, gout), lambda i: (0, 0)),
        ],
        out_specs=[
            pl.BlockSpec((bm, gout), lambda i: (i, 0)),
            pl.BlockSpec((bm, n), lambda i: (i, 0)),
        ],
        out_shape=[
            jax.ShapeDtypeStruct((n, gout), jnp.bfloat16),
            jax.ShapeDtypeStruct((n, n), jnp.float8_e4m3fn),
        ],
        compiler_params=pltpu.CompilerParams(
            dimension_semantics=("parallel",)),
    )(adj, h, b.reshape(1, -1), w_next)


def _layer_body(adj_ref, h_ref, b_ref, *rest, relu, fused):
    if fused:
        wn_ref, o_ref = rest
    else:
        (o_ref,) = rest
    y = jnp.dot(adj_ref[...].astype(jnp.bfloat16), h_ref[...],
                preferred_element_type=jnp.float32)
    y = y + b_ref[...]
    if relu:
        y = jnp.maximum(y, 0.0)
    if fused:
        y = jnp.dot(y, wn_ref[...], preferred_element_type=jnp.float32)
    o_ref[...] = y.astype(o_ref.dtype)


def _layer(adj16, h, b, w_next=None, relu=True, out_dtype=jnp.float32):
    n = adj16.shape[0]
    g = h.shape[1]
    gout = w_next.shape[1] if w_next is not None else g
    bm = _row_tile(n, 400)
    fused = w_next is not None
    args = [adj16, h, b.reshape(1, -1)]
    in_specs = [
        pl.BlockSpec((bm, n), lambda i: (i, 0)),
        pl.BlockSpec((n, g), lambda i: (0, 0)),
        pl.BlockSpec((1, g), lambda i: (0, 0)),
    ]
    if fused:
        args.append(w_next)
        in_specs.append(pl.BlockSpec((g, gout), lambda i: (0, 0)))
    return pl.pallas_call(
        functools.partial(_layer_body, relu=relu, fused=fused),
        grid=(n // bm,),
        in_specs=in_specs,
        out_specs=pl.BlockSpec((bm, gout), lambda i: (i, 0)),
        out_shape=jax.ShapeDtypeStruct((n, gout), out_dtype),
        compiler_params=pltpu.CompilerParams(
            dimension_semantics=("parallel",)),
    )(*args)


def _gram_body(z_ref, zt_ref, o_ref):
    o_ref[...] = jnp.dot(z_ref[...], zt_ref[...], preferred_element_type=jnp.float32)


def _gram(z):
    n, g = z.shape
    bm = _row_tile(n, 400)
    zt = z.T
    return pl.pallas_call(
        _gram_body,
        grid=(n // bm,),
        in_specs=[
            pl.BlockSpec((bm, g), lambda i: (i, 0)),
            pl.BlockSpec((g, n), lambda i: (0, 0)),
        ],
        out_specs=pl.BlockSpec((bm, n), lambda i: (i, 0)),
        out_shape=jax.ShapeDtypeStruct((n, n), jnp.float32),
        compiler_params=pltpu.CompilerParams(
            dimension_semantics=("parallel",)),
    )(z, zt)


def kernel(feat, adj, W1, b1, W2, b2, W3, b3):
    h1 = _input_proj(feat, W1)
    h2, adj16 = _layer1(adj, h1, b1, W2)
    h3 = _layer(adj16, h2, b2, w_next=W3, relu=True, out_dtype=jnp.bfloat16)
    z = _layer(adj16, h3, b3, w_next=None, relu=False, out_dtype=jnp.float32)
    return _gram(z)


# speedup vs baseline: 1.2396x; 1.0032x over previous
import functools

import jax
import jax.numpy as jnp
from jax.experimental import pallas as pl
from jax.experimental.pallas import tpu as pltpu


def _row_tile(n: int, target: int) -> int:
    for t in range(target, 0, -1):
        if n % t == 0 and t % 8 == 0:
            return t
    return n


def _matmul_body(x_ref, w_ref, o_ref):
    h = jnp.dot(x_ref[...], w_ref[...], preferred_element_type=jnp.float32)
    o_ref[...] = h.astype(o_ref.dtype)


def _input_proj(x, w):
    n = x.shape[0]
    g = w.shape[1]
    return pl.pallas_call(
        _matmul_body,
        out_shape=jax.ShapeDtypeStruct((n, g), jnp.bfloat16),
    )(x, w)


def _layer1_body(adj_ref, h_ref, b_ref, wn_ref, o_ref, adj8_ref):
    a16 = adj_ref[...].astype(jnp.bfloat16)
    adj8_ref[...] = adj_ref[...].astype(jnp.float8_e4m3fn)
    y = jnp.dot(a16, h_ref[...], preferred_element_type=jnp.float32)
    y = jnp.maximum(y + b_ref[...], 0.0)
    h2 = jnp.dot(y, wn_ref[...], preferred_element_type=jnp.float32)
    o_ref[...] = h2.astype(jnp.bfloat16)


def _layer1(adj, h, b, w_next):
    n = adj.shape[0]
    g = h.shape[1]
    gout = w_next.shape[1]
    bm = _row_tile(n, 400)
    return pl.pallas_call(
        _layer1_body,
        grid=(n // bm,),
        in_specs=[
            pl.BlockSpec((bm, n), lambda i: (i, 0)),
            pl.BlockSpec((n, g), lambda i: (0, 0)),
            pl.BlockSpec((1, g), lambda i: (0, 0)),
            pl.BlockSpec((g, gout), lambda i: (0, 0)),
        ],
        out_specs=[
            pl.BlockSpec((bm, gout), lambda i: (i, 0)),
            pl.BlockSpec((bm, n), lambda i: (i, 0)),
        ],
        out_shape=[
            jax.ShapeDtypeStruct((n, gout), jnp.bfloat16),
            jax.ShapeDtypeStruct((n, n), jnp.float8_e4m3fn),
        ],
        compiler_params=pltpu.CompilerParams(
            dimension_semantics=("parallel",)),
    )(adj, h, b.reshape(1, -1), w_next)


def _layer_body(adj_ref, h_ref, b_ref, *rest, relu, fused):
    if fused:
        wn_ref, o_ref = rest
    else:
        (o_ref,) = rest
    y = jnp.dot(adj_ref[...], h_ref[...],
                preferred_element_type=jnp.float32)
    y = y + b_ref[...]
    if relu:
        y = jnp.maximum(y, 0.0)
    if fused:
        y = jnp.dot(y, wn_ref[...], preferred_element_type=jnp.float32)
    o_ref[...] = y.astype(o_ref.dtype)


def _layer(adj16, h, b, w_next=None, relu=True, out_dtype=jnp.float32):
    n = adj16.shape[0]
    g = h.shape[1]
    gout = w_next.shape[1] if w_next is not None else g
    bm = _row_tile(n, 1000)
    fused = w_next is not None
    args = [adj16, h, b.reshape(1, -1)]
    in_specs = [
        pl.BlockSpec((bm, n), lambda i: (i, 0)),
        pl.BlockSpec((n, g), lambda i: (0, 0)),
        pl.BlockSpec((1, g), lambda i: (0, 0)),
    ]
    if fused:
        args.append(w_next)
        in_specs.append(pl.BlockSpec((g, gout), lambda i: (0, 0)))
    return pl.pallas_call(
        functools.partial(_layer_body, relu=relu, fused=fused),
        grid=(n // bm,),
        in_specs=in_specs,
        out_specs=pl.BlockSpec((bm, gout), lambda i: (i, 0)),
        out_shape=jax.ShapeDtypeStruct((n, gout), out_dtype),
        compiler_params=pltpu.CompilerParams(
            dimension_semantics=("parallel",)),
    )(*args)


def _gram_body(z_ref, zt_ref, o_ref):
    o_ref[...] = jnp.dot(z_ref[...], zt_ref[...], preferred_element_type=jnp.float32)


def _gram(z):
    n, g = z.shape
    bm = _row_tile(n, 400)
    zt = z.T
    return pl.pallas_call(
        _gram_body,
        grid=(n // bm,),
        in_specs=[
            pl.BlockSpec((bm, g), lambda i: (i, 0)),
            pl.BlockSpec((g, n), lambda i: (0, 0)),
        ],
        out_specs=pl.BlockSpec((bm, n), lambda i: (i, 0)),
        out_shape=jax.ShapeDtypeStruct((n, n), jnp.float32),
        compiler_params=pltpu.CompilerParams(
            dimension_semantics=("parallel",)),
    )(z, zt)


def kernel(feat, adj, W1, b1, W2, b2, W3, b3):
    h1 = _input_proj(feat, W1)
    h2, adj16 = _layer1(adj, h1, b1, W2)
    h3 = _layer(adj16, h2, b2, w_next=W3, relu=True, out_dtype=jnp.bfloat16)
    z = _layer(adj16, h3, b3, w_next=None, relu=False, out_dtype=jnp.float32)
    return _gram(z)
